# baseline (speedup 1.0000x reference)
import functools

import jax
import jax.numpy as jnp
from jax import lax
from jax.experimental import pallas as pl
from jax.experimental.pallas import tpu as pltpu
from jax.experimental.pallas import tpu_sc as plsc

D32 = 32
WSUB = 4
NSUB = WSUB + 3
NC, NS = 2, 16
NW = NC * NS
NBUF = 2


@functools.partial(jax.jit, static_argnames=("T", "C", "nchunk"))
def _emb_call(idx, table, T, C, nchunk):
    tpw = T // NW
    mesh = plsc.VectorSubcoreMesh(core_axis_name="c", subcore_axis_name="s")

    buf_types = []
    for _ in range(NBUF):
        buf_types += [
            pltpu.VMEM((NSUB * C, D32), jnp.float32),
            pltpu.SemaphoreType.DMA,
            pltpu.SemaphoreType.DMA,
        ]

    @functools.partial(
        pl.kernel,
        out_type=jax.ShapeDtypeStruct((T * NSUB, D32), jnp.float32),
        mesh=mesh,
        scratch_types=[pltpu.VMEM((nchunk, NSUB * C), jnp.int32)] + buf_types,
        compiler_params=pltpu.CompilerParams(use_tc_tiling_on_sc=False),
    )
    def emb(idx_hbm, tbl_hbm, out_hbm, idx_v, *bufs):
        slots = [bufs[3 * b:3 * b + 3] for b in range(NBUF)]
        wid = lax.axis_index("s") * NC + lax.axis_index("c")
        pltpu.sync_copy(idx_hbm.at[wid], idx_v)

        def fire(i, b):
            buf, gsem, _ = slots[b]
            pltpu.async_copy(tbl_hbm.at[idx_v.at[i]], buf, gsem)

        def drain_gather_fire_write(i, b):
            buf, gsem, wsem = slots[b]
            pltpu.make_async_copy(tbl_hbm.at[idx_v.at[i]], buf, gsem).wait()
            base = (wid * tpw + i * C) * NSUB
            pltpu.async_copy(buf, out_hbm.at[pl.ds(base, NSUB * C)], wsem)

        def drain_write(i, b):
            buf, _, wsem = slots[b]
            base = (wid * tpw + i * C) * NSUB
            pltpu.make_async_copy(buf, out_hbm.at[pl.ds(base, NSUB * C)], wsem).wait()

        for b in range(NBUF):
            fire(b, b)

        @pl.loop(0, nchunk // NBUF)
        def body(j):
            for b in range(NBUF):
                i = j * NBUF + b
                drain_gather_fire_write(i, b)

                @pl.when(i + NBUF < nchunk)
                def _():
                    drain_write(i, b)
                    fire(i + NBUF, b)

                @pl.when(i + NBUF >= nchunk)
                def _():
                    drain_write(i, b)

    return emb(idx, table)


def kernel(word_id, tag_id, pos_1, pos_2, word_table, tag_table, pos_table):
    B, L = word_id.shape
    T = B * L
    C = 128
    nchunk = T // (NW * C)
    ntag = tag_table.shape[0]
    nword_sub = word_table.shape[0] * WSUB
    table = jnp.concatenate(
        [word_table.reshape(-1, D32), tag_table, pos_table], axis=0)
    idx = jnp.concatenate([
        word_id.reshape(T, 1).astype(jnp.int32) * WSUB
        + jnp.arange(WSUB, dtype=jnp.int32),
        tag_id.reshape(T, 1).astype(jnp.int32) + nword_sub,
        pos_1.reshape(T, 1).astype(jnp.int32) + (nword_sub + ntag),
        pos_2.reshape(T, 1).astype(jnp.int32) + (nword_sub + ntag),
    ], axis=-1)
    out = _emb_call(
        idx.reshape(NW, nchunk, NSUB * C), table,
        T=T, C=C, nchunk=nchunk,
    )
    return out.reshape(B, L, NSUB * D32)


# speedup vs baseline: 1.8858x; 1.8858x over previous
import functools

import jax
import jax.numpy as jnp
from jax import lax
from jax.experimental import pallas as pl
from jax.experimental.pallas import tpu as pltpu
from jax.experimental.pallas import tpu_sc as plsc

WD, D32 = 128, 32
TPSUB = 3
NC, NS = 2, 16
NW = NC * NS
NBUF = 3


@functools.partial(jax.jit, static_argnames=("T", "C", "nchunk"))
def _emb_call(widx, tpidx, word_table, tp_table, T, C, nchunk):
    tpw = T // NW
    mesh = plsc.VectorSubcoreMesh(core_axis_name="c", subcore_axis_name="s")

    buf_types = []
    for _ in range(NBUF):
        buf_types += [
            pltpu.VMEM((C, WD), jnp.float32),
            pltpu.VMEM((TPSUB * C, D32), jnp.float32),
            pltpu.SemaphoreType.DMA,
            pltpu.SemaphoreType.DMA,
        ]

    @functools.partial(
        pl.kernel,
        out_type=jax.ShapeDtypeStruct((T, WD + TPSUB * D32), jnp.float32),
        mesh=mesh,
        scratch_types=[
            pltpu.VMEM((nchunk, C), jnp.int32),
            pltpu.VMEM((nchunk, TPSUB * C), jnp.int32),
        ] + buf_types,
        compiler_params=pltpu.CompilerParams(use_tc_tiling_on_sc=False),
    )
    def emb(wi_hbm, tpi_hbm, wt_hbm, tp_hbm, out_hbm, widx_v, tpidx_v, *bufs):
        slots = [bufs[4 * b:4 * b + 4] for b in range(NBUF)]
        wid = lax.axis_index("s") * NC + lax.axis_index("c")
        pltpu.sync_copy(wi_hbm.at[wid], widx_v)
        pltpu.sync_copy(tpi_hbm.at[wid], tpidx_v)

        def fire_gathers(i, b):
            wbuf, tpbuf, gsem, _ = slots[b]
            pltpu.async_copy(wt_hbm.at[widx_v.at[i]], wbuf, gsem)
            pltpu.async_copy(tp_hbm.at[tpidx_v.at[i]], tpbuf, gsem)

        def drain_gathers(i, b):
            wbuf, tpbuf, gsem, _ = slots[b]
            pltpu.make_async_copy(wt_hbm.at[widx_v.at[i]], wbuf, gsem).wait()
            pltpu.make_async_copy(tp_hbm.at[tpidx_v.at[i]], tpbuf, gsem).wait()

        def fire_writes(i, b):
            wbuf, tpbuf, _, wsem = slots[b]
            base = wid * tpw + i * C
            pltpu.async_copy(wbuf, out_hbm.at[pl.ds(base, C), pl.ds(0, WD)], wsem)
            for k in range(TPSUB):
                pltpu.async_copy(
                    tpbuf.at[pl.ds(k * C, C)],
                    out_hbm.at[pl.ds(base, C), pl.ds(WD + k * D32, D32)], wsem)

        def drain_writes(i, b):
            wbuf, tpbuf, _, wsem = slots[b]
            base = wid * tpw + i * C
            pltpu.make_async_copy(
                wbuf, out_hbm.at[pl.ds(base, C), pl.ds(0, WD)], wsem).wait()
            for k in range(TPSUB):
                pltpu.make_async_copy(
                    tpbuf.at[pl.ds(k * C, C)],
                    out_hbm.at[pl.ds(base, C), pl.ds(WD + k * D32, D32)],
                    wsem).wait()

        def step(i, b, first=False, fire_ahead=True):
            if not first:
                drain_writes(i - 1, (b - 1) % NBUF)
            if fire_ahead:
                fire_gathers(i + 2, (b + 2) % NBUF)
            drain_gathers(i, b)
            fire_writes(i, b)

        fire_gathers(0, 0)
        fire_gathers(1, 1)

        nmain = nchunk - (nchunk % NBUF)

        @pl.loop(0, nmain // NBUF)
        def body(j):
            for b in range(NBUF):
                i = j * NBUF + b

                @pl.when(i == 0)
                def _():
                    step(i, b, first=True)

                @pl.when(jnp.logical_and(i > 0, i + 2 < nchunk))
                def _():
                    step(i, b)

                @pl.when(jnp.logical_and(i > 0, i + 2 >= nchunk))
                def _():
                    step(i, b, fire_ahead=False)

        for i in range(nmain, nchunk):
            step(i, i % NBUF, fire_ahead=(i + 2 < nchunk))
        drain_writes(nchunk - 1, (nchunk - 1) % NBUF)

    return emb(widx, tpidx, word_table, tp_table)


def kernel(word_id, tag_id, pos_1, pos_2, word_table, tag_table, pos_table):
    B, L = word_id.shape
    T = B * L
    C = 128
    nchunk = T // (NW * C)
    ntag = tag_table.shape[0]
    tp_table = jnp.concatenate([tag_table, pos_table], axis=0)
    tpidx = jnp.stack([
        tag_id.reshape(NW, nchunk, C).astype(jnp.int32),
        pos_1.reshape(NW, nchunk, C).astype(jnp.int32) + ntag,
        pos_2.reshape(NW, nchunk, C).astype(jnp.int32) + ntag,
    ], axis=2)
    out = _emb_call(
        word_id.reshape(NW, nchunk, C).astype(jnp.int32),
        tpidx.reshape(NW, nchunk, TPSUB * C),
        word_table, tp_table,
        T=T, C=C, nchunk=nchunk,
    )
    return out.reshape(B, L, WD + TPSUB * D32)
